# baseline (speedup 1.0000x reference)
import jax
import jax.numpy as jnp
from jax.experimental import pallas as pl

_BLOCK_ROWS = 4096
_LN2 = 0.6931471805599453


def _yj_body(x_ref, v_ref, o_ref):
    x = x_ref[...]
    p1 = v_ref[0:1, :]
    p2 = v_ref[1:2, :]
    q1 = v_ref[2:3, :]
    q2 = v_ref[3:4, :]
    pos = x >= 0.0
    t2 = jnp.log2(1.0 + jnp.abs(x))
    c = jnp.where(pos, p1, p2)
    em1 = jnp.exp2(c * t2) - 1.0
    a = jnp.where(c == 0.0, t2, em1)
    m = jnp.where(pos, q1, q2)
    o_ref[...] = a * m


def kernel(x, lmbda):
    n, d = x.shape
    p1 = lmbda.reshape(1, d)
    p2 = 2.0 - p1
    q1 = jnp.where(p1 == 0.0, _LN2, 1.0 / jnp.where(p1 == 0.0, 1.0, p1))
    q2 = jnp.where(p2 == 0.0, -_LN2, -1.0 / jnp.where(p2 == 0.0, 1.0, p2))
    v = jnp.concatenate([p1, p2, q1, q2], axis=0)
    grid = (n // _BLOCK_ROWS,)
    return pl.pallas_call(
        _yj_body,
        grid=grid,
        in_specs=[
            pl.BlockSpec((_BLOCK_ROWS, d), lambda i: (i, 0)),
            pl.BlockSpec((4, d), lambda i: (0, 0)),
        ],
        out_specs=pl.BlockSpec((_BLOCK_ROWS, d), lambda i: (i, 0)),
        out_shape=jax.ShapeDtypeStruct((n, d), x.dtype),
    )(x, v)


# speedup vs baseline: 1.0226x; 1.0226x over previous
import jax
import jax.numpy as jnp
from jax.experimental import pallas as pl
from jax.experimental.pallas import tpu as pltpu

_N, _D = 65536, 512
_BLOCK_ROWS = 4096


_LN2 = 0.6931471805599453


def _yj_body(x_ref, lm_ref, o_ref, p2_ref, q1_ref, q2_ref):
    @pl.when(pl.program_id(0) == 0)
    def _init():
        lm = lm_ref[...]
        p2_ref[...] = 2.0 - lm
        q1_ref[...] = jnp.where(lm == 0.0, _LN2, 1.0 / jnp.where(lm == 0.0, 1.0, lm))
        q2_ref[...] = jnp.where(
            lm == 2.0, -_LN2, -1.0 / jnp.where(lm == 2.0, 1.0, 2.0 - lm)
        )

    x = x_ref[...]
    p1 = lm_ref[...]
    p2 = p2_ref[...]
    q1 = q1_ref[...]
    q2 = q2_ref[...]
    pos = x >= 0.0
    t2 = jnp.log2(1.0 + jnp.abs(x))
    c = jnp.where(pos, p1, p2)
    em1 = jnp.exp2(c * t2) - 1.0
    a = jnp.where(c == 0.0, t2, em1)
    m = jnp.where(pos, q1, q2)
    o_ref[...] = a * m


def kernel(x, lmbda):
    n, d = x.shape
    lm2 = lmbda.reshape(1, d)
    grid = (n // _BLOCK_ROWS,)
    return pl.pallas_call(
        _yj_body,
        grid=grid,
        in_specs=[
            pl.BlockSpec((_BLOCK_ROWS, d), lambda i: (i, 0)),
            pl.BlockSpec((1, d), lambda i: (0, 0)),
        ],
        out_specs=pl.BlockSpec((_BLOCK_ROWS, d), lambda i: (i, 0)),
        out_shape=jax.ShapeDtypeStruct((n, d), x.dtype),
        scratch_shapes=[pltpu.VMEM((1, d), jnp.float32)] * 3,
    )(x, lm2)


# speedup vs baseline: 1.0963x; 1.0722x over previous
import jax
import jax.numpy as jnp
from jax import lax
from jax.experimental import pallas as pl
from jax.experimental.pallas import tpu as pltpu

_CH = 2048
_NBUF = 4
_LN2 = 0.6931471805599453


def _yj(x, p1, p2, q1, q2):
    pos = x >= 0.0
    t2 = jnp.log2(1.0 + jnp.abs(x))
    c = jnp.where(pos, p1, p2)
    em1 = jnp.exp2(c * t2) - 1.0
    a = jnp.where(c == 0.0, t2, em1)
    m = jnp.where(pos, q1, q2)
    return a * m


def _body(x_hbm, lm_ref, o_hbm, in_buf, out_buf, in_sems, out_sems):
    n = x_hbm.shape[0]
    nchunk = n // _CH
    lm = lm_ref[...]
    p2 = 2.0 - lm
    q1 = jnp.where(lm == 0.0, _LN2, 1.0 / jnp.where(lm == 0.0, 1.0, lm))
    q2 = jnp.where(lm == 2.0, -_LN2, -1.0 / jnp.where(lm == 2.0, 1.0, p2))

    def in_copy(c, s):
        return pltpu.make_async_copy(
            x_hbm.at[pl.ds(c * _CH, _CH)], in_buf.at[s], in_sems.at[s]
        )

    def out_copy(c, s):
        return pltpu.make_async_copy(
            out_buf.at[s], o_hbm.at[pl.ds(c * _CH, _CH)], out_sems.at[s]
        )

    for s in range(_NBUF):
        in_copy(s, s).start()

    def step(i, carry):
        s = lax.rem(i, _NBUF)
        in_copy(i, s).wait()

        @pl.when(i >= _NBUF)
        def _wait_out():
            out_copy(i - _NBUF, s).wait()

        out_buf[s] = _yj(in_buf[s], lm, p2, q1, q2)
        out_copy(i, s).start()

        @pl.when(i + _NBUF < nchunk)
        def _next_in():
            in_copy(i + _NBUF, s).start()

        return carry

    lax.fori_loop(0, nchunk, step, 0)

    for k in range(_NBUF):
        c = nchunk - _NBUF + k
        out_copy(c, c % _NBUF).wait()


def kernel(x, lmbda):
    n, d = x.shape
    lm2 = lmbda.reshape(1, d)
    return pl.pallas_call(
        _body,
        in_specs=[
            pl.BlockSpec(memory_space=pltpu.HBM),
            pl.BlockSpec(memory_space=pltpu.VMEM),
        ],
        out_specs=pl.BlockSpec(memory_space=pltpu.HBM),
        out_shape=jax.ShapeDtypeStruct((n, d), x.dtype),
        scratch_shapes=[
            pltpu.VMEM((_NBUF, _CH, d), jnp.float32),
            pltpu.VMEM((_NBUF, _CH, d), jnp.float32),
            pltpu.SemaphoreType.DMA((_NBUF,)),
            pltpu.SemaphoreType.DMA((_NBUF,)),
        ],
    )(x, lm2)
